# Initial kernel scaffold; baseline (speedup 1.0000x reference)
import jax
import jax.numpy as jnp
from jax.experimental import pallas as pl


def kernel(time_images_season_list):
    raise NotImplementedError("write your pallas kernel here")


# speedup vs baseline: 39.7917x; 39.7917x over previous
import jax
import jax.numpy as jnp
from jax.experimental import pallas as pl


_TB = 256


def _masked_copy_kernel(x_ref, o_ref):
    b = pl.program_id(0)

    @pl.when(b == 0)
    def _zero():
        o_ref[...] = jnp.zeros_like(o_ref)

    @pl.when(b != 0)
    def _copy():
        o_ref[...] = x_ref[...]


def kernel(time_images_season_list):
    x = time_images_season_list
    _, b, t, c, n = x.shape
    x2 = x.reshape(b, t, c * n)
    out = pl.pallas_call(
        _masked_copy_kernel,
        grid=(b, t // _TB),
        in_specs=[pl.BlockSpec((1, _TB, c * n), lambda i, j: (i, j, 0))],
        out_specs=pl.BlockSpec((1, _TB, c * n), lambda i, j: (i, j, 0)),
        out_shape=jax.ShapeDtypeStruct((b, t, c * n), x.dtype),
    )(x2)
    return out.reshape(b, t, c, n)


# speedup vs baseline: 51.0308x; 1.2824x over previous
import jax
import jax.numpy as jnp
from jax.experimental import pallas as pl


_TB = 1024


def _masked_copy_kernel(x_ref, o_ref):
    b = pl.program_id(0)

    @pl.when(b == 0)
    def _zero():
        o_ref[...] = jnp.zeros_like(o_ref)

    @pl.when(b != 0)
    def _copy():
        o_ref[...] = x_ref[...]


def kernel(time_images_season_list):
    x = time_images_season_list
    _, b, t, c, n = x.shape
    x2 = x.reshape(b, t, c * n)
    out = pl.pallas_call(
        _masked_copy_kernel,
        grid=(b, t // _TB),
        in_specs=[pl.BlockSpec((1, _TB, c * n), lambda i, j: (i, j, 0))],
        out_specs=pl.BlockSpec((1, _TB, c * n), lambda i, j: (i, j, 0)),
        out_shape=jax.ShapeDtypeStruct((b, t, c * n), x.dtype),
    )(x2)
    return out.reshape(b, t, c, n)


# speedup vs baseline: 52.1581x; 1.0221x over previous
import jax
import jax.numpy as jnp
from jax.experimental import pallas as pl


_TB = 2048


def _masked_copy_kernel(x_ref, o_ref):
    b = pl.program_id(0)

    @pl.when(b == 0)
    def _zero():
        o_ref[...] = jnp.zeros_like(o_ref)

    @pl.when(b != 0)
    def _copy():
        o_ref[...] = x_ref[...]


def kernel(time_images_season_list):
    x = time_images_season_list
    _, b, t, c, n = x.shape
    x2 = x.reshape(b, t, c * n)
    out = pl.pallas_call(
        _masked_copy_kernel,
        grid=(b, t // _TB),
        in_specs=[pl.BlockSpec((1, _TB, c * n), lambda i, j: (i, j, 0))],
        out_specs=pl.BlockSpec((1, _TB, c * n), lambda i, j: (i, j, 0)),
        out_shape=jax.ShapeDtypeStruct((b, t, c * n), x.dtype),
    )(x2)
    return out.reshape(b, t, c, n)
